# baseline (speedup 1.0000x reference)
import functools

import jax
import jax.numpy as jnp
from jax.experimental import pallas as pl
from jax.experimental.pallas import tpu as pltpu

_FOLD = 512


def _chamfer_body(x1_ref, x1a_ref, q2s_ref, n2_ref, x2cat_ref, out_ref,
                  *, nb_size, mb_size, m_total, inv_count):
    b = pl.program_id(0)
    nb = pl.program_id(1)
    last_b = pl.num_programs(0) - 1
    last_nb = pl.num_programs(1) - 1
    n_chunks = m_total // mb_size

    x1 = x1_ref[0]
    x1x = x1[:, 0:1]
    x1y = x1[:, 1:2]
    x1z = x1[:, 2:3]
    x1a = x1a_ref[0]

    scores = []
    best = jnp.full((nb_size, _FOLD), -jnp.inf, dtype=jnp.float32)
    for i in range(n_chunks):
        xs = q2s_ref[0, :, i * mb_size:(i + 1) * mb_size]
        halfn2 = n2_ref[0, 0:1, i * mb_size:(i + 1) * mb_size]
        score = jnp.dot(x1a, xs, preferred_element_type=jnp.float32) + halfn2
        scores.append(score)
        for j in range(mb_size // _FOLD):
            best = jnp.maximum(best, score[:, j * _FOLD:(j + 1) * _FOLD])
    rowmax = jnp.max(best, axis=1, keepdims=True)

    acc = jnp.zeros((nb_size, 8), dtype=jnp.float32)
    for i in range(n_chunks):
        oh = jnp.where(scores[i] == rowmax, 1.0, 0.0).astype(jnp.bfloat16)
        xc = x2cat_ref[0, i * mb_size:(i + 1) * mb_size, :]
        acc = acc + jnp.dot(oh, xc, preferred_element_type=jnp.float32)
    invc = 1.0 / acc[:, 6:7]
    nnx = (acc[:, 0:1] + acc[:, 3:4]) * invc
    nny = (acc[:, 1:2] + acc[:, 4:5]) * invc
    nnz = (acc[:, 2:3] + acc[:, 5:6]) * invc
    tx = x1x - nnx
    ty = x1y - nny
    tz = x1z - nnz
    s = jnp.sum(tx * tx + ty * ty + tz * tz)

    is_first = jnp.logical_and(b == 0, nb == 0)
    prev = jnp.where(is_first, jnp.float32(0.0), out_ref[0, 0])
    total = prev + s
    is_last = jnp.logical_and(b == last_b, nb == last_nb)
    out_ref[0, 0] = jnp.where(is_last, total * inv_count, total)


def kernel(xyz1, xyz2):
    B, N, _ = xyz1.shape
    M = xyz2.shape[1]
    nb_size = min(512, N)
    mb_size = min(2048, M)

    f32 = jnp.float32
    bf16 = jnp.bfloat16

    zeros_n = jnp.zeros((B, N, 13), dtype=bf16)
    x1a = jnp.concatenate([xyz1.astype(bf16), zeros_n], axis=-1)

    x2t = jnp.transpose(xyz2, (0, 2, 1))
    n2h = -0.5 * (x2t[:, 0:1, :] * x2t[:, 0:1, :]
                  + x2t[:, 1:2, :] * x2t[:, 1:2, :]
                  + x2t[:, 2:3, :] * x2t[:, 2:3, :])
    n2row = jnp.concatenate(
        [n2h, jnp.zeros((B, 7, M), dtype=f32)], axis=1)
    zeros_m = jnp.zeros((B, 13, M), dtype=bf16)
    q2s = jnp.concatenate([x2t.astype(bf16), zeros_m], axis=1)

    hi = xyz2.astype(bf16)
    lo = (xyz2 - hi.astype(f32)).astype(bf16)
    ones_m = jnp.ones((B, M, 1), dtype=bf16)
    zero_m = jnp.zeros((B, M, 1), dtype=bf16)
    x2cat = jnp.concatenate([hi, lo, ones_m, zero_m], axis=-1)

    body = functools.partial(
        _chamfer_body,
        nb_size=nb_size,
        mb_size=mb_size,
        m_total=M,
        inv_count=1.0 / (B * N),
    )
    out = pl.pallas_call(
        body,
        grid=(B, N // nb_size),
        in_specs=[
            pl.BlockSpec((1, nb_size, 3), lambda b, nb: (b, nb, 0)),
            pl.BlockSpec((1, nb_size, 16), lambda b, nb: (b, nb, 0)),
            pl.BlockSpec((1, 16, M), lambda b, nb: (b, 0, 0)),
            pl.BlockSpec((1, 8, M), lambda b, nb: (b, 0, 0)),
            pl.BlockSpec((1, M, 8), lambda b, nb: (b, 0, 0)),
        ],
        out_specs=pl.BlockSpec(memory_space=pltpu.SMEM),
        out_shape=jax.ShapeDtypeStruct((1, 1), jnp.float32),
    )(xyz1, x1a, q2s, n2row, x2cat)
    return out[0, 0]


# speedup vs baseline: 1.3382x; 1.3382x over previous
import functools

import jax
import jax.numpy as jnp
from jax.experimental import pallas as pl
from jax.experimental.pallas import tpu as pltpu

_FOLD = 512


def _chamfer_body(x1_ref, q2s_ref, x2cat_ref, out_ref,
                  *, nb_size, mb_size, m_total, inv_count):
    b = pl.program_id(0)
    nb = pl.program_id(1)
    last_b = pl.num_programs(0) - 1
    last_nb = pl.num_programs(1) - 1
    n_chunks = m_total // mb_size

    x1 = x1_ref[0]
    x1x = x1[:, 0:1]
    x1y = x1[:, 1:2]
    x1z = x1[:, 2:3]

    scores = []
    best = jnp.full((nb_size, _FOLD), -jnp.inf, dtype=jnp.float32)
    for i in range(n_chunks):
        xs = q2s_ref[0, :, i * mb_size:(i + 1) * mb_size]
        score = (x1x * xs[0:1, :] + xs[3:4, :]
                 + x1y * xs[1:2, :]
                 + x1z * xs[2:3, :])
        scores.append(score)
        for j in range(mb_size // _FOLD):
            best = jnp.maximum(best, score[:, j * _FOLD:(j + 1) * _FOLD])
    rowmax = jnp.max(best, axis=1, keepdims=True)

    acc = jnp.zeros((nb_size, 8), dtype=jnp.float32)
    for i in range(n_chunks):
        oh = jnp.where(scores[i] == rowmax, 1.0, 0.0).astype(jnp.bfloat16)
        xc = x2cat_ref[0, i * mb_size:(i + 1) * mb_size, :]
        acc = acc + jnp.dot(oh, xc, preferred_element_type=jnp.float32)
    invc = 1.0 / acc[:, 6:7]
    nnx = (acc[:, 0:1] + acc[:, 3:4]) * invc
    nny = (acc[:, 1:2] + acc[:, 4:5]) * invc
    nnz = (acc[:, 2:3] + acc[:, 5:6]) * invc
    tx = x1x - nnx
    ty = x1y - nny
    tz = x1z - nnz
    s = jnp.sum(tx * tx + ty * ty + tz * tz)

    is_first = jnp.logical_and(b == 0, nb == 0)
    prev = jnp.where(is_first, jnp.float32(0.0), out_ref[0, 0])
    total = prev + s
    is_last = jnp.logical_and(b == last_b, nb == last_nb)
    out_ref[0, 0] = jnp.where(is_last, total * inv_count, total)


def kernel(xyz1, xyz2):
    B, N, _ = xyz1.shape
    M = xyz2.shape[1]
    nb_size = min(1024, N)
    mb_size = min(2048, M)

    f32 = jnp.float32
    bf16 = jnp.bfloat16

    x2t = jnp.transpose(xyz2, (0, 2, 1))
    u = jax.lax.bitcast_convert_type(x2t, jnp.uint32)
    r = (u + jnp.uint32(0x7FFF) + ((u >> 16) & jnp.uint32(1))) & jnp.uint32(0xFFFF0000)
    q2t = jax.lax.bitcast_convert_type(r, f32)
    n2h = -0.5 * (x2t[:, 0:1, :] * x2t[:, 0:1, :]
                  + x2t[:, 1:2, :] * x2t[:, 1:2, :]
                  + x2t[:, 2:3, :] * x2t[:, 2:3, :])
    zeros_m = jnp.zeros((B, 4, M), dtype=f32)
    q2s = jnp.concatenate([q2t, n2h, zeros_m], axis=1)

    hi = xyz2.astype(bf16)
    lo = (xyz2 - hi.astype(f32)).astype(bf16)
    ones_m = jnp.ones((B, M, 1), dtype=bf16)
    zero_m = jnp.zeros((B, M, 1), dtype=bf16)
    x2cat = jnp.concatenate([hi, lo, ones_m, zero_m], axis=-1)

    body = functools.partial(
        _chamfer_body,
        nb_size=nb_size,
        mb_size=mb_size,
        m_total=M,
        inv_count=1.0 / (B * N),
    )
    out = pl.pallas_call(
        body,
        grid=(B, N // nb_size),
        in_specs=[
            pl.BlockSpec((1, nb_size, 3), lambda b, nb: (b, nb, 0)),
            pl.BlockSpec((1, 8, M), lambda b, nb: (b, 0, 0)),
            pl.BlockSpec((1, M, 8), lambda b, nb: (b, 0, 0)),
        ],
        out_specs=pl.BlockSpec(memory_space=pltpu.SMEM),
        out_shape=jax.ShapeDtypeStruct((1, 1), jnp.float32),
    )(xyz1, q2s, x2cat)
    return out[0, 0]
